# Initial kernel scaffold; baseline (speedup 1.0000x reference)
import jax
import jax.numpy as jnp
from jax.experimental import pallas as pl


def kernel(x, edge_index, W1_l, W1_r, b1, W2_l, W2_r, b2):
    raise NotImplementedError("write your pallas kernel here")


# speedup vs baseline: 5.5445x; 5.5445x over previous
import functools

import jax
import jax.numpy as jnp
from jax import lax
from jax.experimental import pallas as pl
from jax.experimental.pallas import tpu as pltpu
from jax.experimental.pallas import tpu_sc as plsc

N_NODES = 10000
N_PAD = 10240
D_FEAT = 128
HIDDEN = 128
N_CLASSES = 40
N_EDGES = 320000
CHUNK = 80


@functools.lru_cache(maxsize=None)
def _make_sc_segment_sum(d: int, with_counts: bool):
    info = plsc.get_sparse_core_info()
    nc, ns = info.num_cores, info.num_subcores
    nw = nc * ns
    e_per_w = N_EDGES // nw
    assert N_EDGES % nw == 0 and e_per_w % CHUNK == 0
    n_chunks = e_per_w // CHUNK
    rows_per_s = N_PAD // ns
    zcopies = rows_per_s // CHUNK
    dcol = d // 16

    out_type = [jax.ShapeDtypeStruct((nc, N_PAD, d), jnp.float32)]
    scratch = [
        pltpu.VMEM((CHUNK, d), jnp.float32),
        pltpu.VMEM((CHUNK,), jnp.int32),
        pltpu.VMEM((CHUNK,), jnp.int32),
        pltpu.VMEM_SHARED((N_PAD, d), jnp.float32),
        pltpu.SemaphoreType.DMA,
    ]
    if with_counts:
        out_type.append(jax.ShapeDtypeStruct((nc, ns, N_PAD), jnp.float32))
        scratch.append(pltpu.VMEM((N_PAD,), jnp.float32))

    mesh = plsc.VectorSubcoreMesh(core_axis_name="c", subcore_axis_name="s")

    @functools.partial(
        pl.kernel, mesh=mesh, out_type=tuple(out_type),
        scratch_types=scratch,
        compiler_params=pltpu.CompilerParams(needs_layout_passes=False))
    def k(table_hbm, src_hbm, dst_hbm, *refs):
        if with_counts:
            agg_out, cnt_out, rowbuf, sidx, didx, acc, sem, cntbuf = refs
        else:
            agg_out, rowbuf, sidx, didx, acc, sem = refs
            cnt_out = cntbuf = None
        cid = lax.axis_index("c")
        sid = lax.axis_index("s")
        wid = sid * nc + cid
        row0 = sid * rows_per_s

        def zrow(i, c):
            rowbuf[i // dcol, pl.ds((i % dcol) * 16, 16)] = jnp.zeros(
                (16,), jnp.float32)
            return c
        lax.fori_loop(0, CHUNK * dcol, zrow, 0)
        for t in range(zcopies):
            pltpu.sync_copy(rowbuf, acc.at[pl.ds(row0 + t * CHUNK, CHUNK)])
        if with_counts:
            def zcnt(i, c):
                cntbuf[pl.ds(i * 16, 16)] = jnp.zeros((16,), jnp.float32)
                return c
            lax.fori_loop(0, N_PAD // 16, zcnt, 0)
        plsc.subcore_barrier()

        ebase = wid * e_per_w

        def chunk_body(t, c):
            off = pl.multiple_of(ebase + t * CHUNK, 8)
            pltpu.sync_copy(src_hbm.at[pl.ds(off, CHUNK)], sidx)
            pltpu.sync_copy(dst_hbm.at[pl.ds(off, CHUNK)], didx)
            pltpu.async_copy(table_hbm.at[sidx], rowbuf, sem).wait()
            pltpu.sync_copy(rowbuf, acc.at[didx], add=True)
            if with_counts:
                for j in range(CHUNK // 16):
                    dv = didx[pl.ds(j * 16, 16)]
                    plsc.addupdate_scatter(
                        cntbuf, [dv], jnp.full((16,), 1.0, jnp.float32))
            return c
        lax.fori_loop(0, n_chunks, chunk_body, 0)
        plsc.subcore_barrier()

        pltpu.sync_copy(acc.at[pl.ds(row0, rows_per_s)],
                        agg_out.at[cid, pl.ds(row0, rows_per_s)])
        if with_counts:
            pltpu.sync_copy(cntbuf, cnt_out.at[cid, sid])

    return k


_BR = 1000


def _tc_mid(aggp, cntp, x, w1l, w1r, b1, w2r, b2):
    nb = N_NODES // _BR

    def body(aggp_ref, cntp_ref, x_ref, w1l_ref, w1r_ref, b1_ref,
             w2r_ref, b2_ref, h_ref, z_ref):
        agg = aggp_ref[0] + aggp_ref[1]
        cnt = jnp.sum(cntp_ref[...], axis=1)[:, None]
        mean = agg / jnp.maximum(cnt, 1.0)
        h = jnp.maximum(
            jnp.dot(mean, w1l_ref[...], preferred_element_type=jnp.float32)
            + jnp.dot(x_ref[...], w1r_ref[...],
                      preferred_element_type=jnp.float32)
            + b1_ref[...], 0.0)
        h_ref[...] = h
        z_ref[...] = jnp.dot(h, w2r_ref[...],
                             preferred_element_type=jnp.float32) + b2_ref[...]

    nsub = cntp.shape[1]
    return pl.pallas_call(
        body,
        grid=(nb,),
        in_specs=[
            pl.BlockSpec((2, _BR, D_FEAT), lambda i: (0, i, 0)),
            pl.BlockSpec((_BR, nsub), lambda i: (i, 0)),
            pl.BlockSpec((_BR, D_FEAT), lambda i: (i, 0)),
            pl.BlockSpec((D_FEAT, HIDDEN), lambda i: (0, 0)),
            pl.BlockSpec((D_FEAT, HIDDEN), lambda i: (0, 0)),
            pl.BlockSpec((1, HIDDEN), lambda i: (0, 0)),
            pl.BlockSpec((HIDDEN, N_CLASSES), lambda i: (0, 0)),
            pl.BlockSpec((1, N_CLASSES), lambda i: (0, 0)),
        ],
        out_specs=[
            pl.BlockSpec((_BR, HIDDEN), lambda i: (i, 0)),
            pl.BlockSpec((_BR, N_CLASSES), lambda i: (i, 0)),
        ],
        out_shape=[
            jax.ShapeDtypeStruct((N_NODES, HIDDEN), jnp.float32),
            jax.ShapeDtypeStruct((N_NODES, N_CLASSES), jnp.float32),
        ],
    )(aggp, cntp, x, w1l, w1r, b1, w2r, b2)


def _tc_final(agg2p, cntp, z, w2l):
    nb = N_NODES // _BR

    def body(a2_ref, cntp_ref, z_ref, w2l_ref, o_ref):
        s = a2_ref[0] + a2_ref[1]
        cnt = jnp.sum(cntp_ref[...], axis=1)[:, None]
        mean = s / jnp.maximum(cnt, 1.0)
        o_ref[...] = jnp.dot(mean, w2l_ref[...],
                             preferred_element_type=jnp.float32) + z_ref[...]

    nsub = cntp.shape[1]
    return pl.pallas_call(
        body,
        grid=(nb,),
        in_specs=[
            pl.BlockSpec((2, _BR, HIDDEN), lambda i: (0, i, 0)),
            pl.BlockSpec((_BR, nsub), lambda i: (i, 0)),
            pl.BlockSpec((_BR, N_CLASSES), lambda i: (i, 0)),
            pl.BlockSpec((HIDDEN, N_CLASSES), lambda i: (0, 0)),
        ],
        out_specs=pl.BlockSpec((_BR, N_CLASSES), lambda i: (i, 0)),
        out_shape=jax.ShapeDtypeStruct((N_NODES, N_CLASSES), jnp.float32),
    )(agg2p, cntp, z, w2l)


def kernel(x, edge_index, W1_l, W1_r, b1, W2_l, W2_r, b2):
    src = edge_index[0].astype(jnp.int32)
    dst = edge_index[1].astype(jnp.int32)

    sc1 = _make_sc_segment_sum(D_FEAT, True)
    aggp, cntp = sc1(x, src, dst)
    cntp = cntp.reshape(-1, N_PAD).T

    h, z = _tc_mid(aggp, cntp, x, W1_l, W1_r, b1.reshape(1, -1),
                   W2_r, b2.reshape(1, -1))

    sc2 = _make_sc_segment_sum(HIDDEN, False)
    agg2p = sc2(h, src, dst)
    if isinstance(agg2p, (tuple, list)):
        agg2p = agg2p[0]

    return _tc_final(agg2p, cntp, z, W2_l)


# speedup vs baseline: 12.0886x; 2.1803x over previous
import functools

import jax
import jax.numpy as jnp
from jax import lax
from jax.experimental import pallas as pl
from jax.experimental.pallas import tpu as pltpu
from jax.experimental.pallas import tpu_sc as plsc

N_NODES = 10000
N_PAD = 10240
D_FEAT = 128
HIDDEN = 128
N_CLASSES = 40
N_EDGES = 320000
CHUNK = 80


@functools.lru_cache(maxsize=None)
def _make_sc_segment_sum(d: int, with_counts: bool):
    info = plsc.get_sparse_core_info()
    nc, ns = info.num_cores, info.num_subcores
    nw = nc * ns
    e_per_w = N_EDGES // nw
    assert N_EDGES % nw == 0 and e_per_w % CHUNK == 0
    n_chunks = e_per_w // CHUNK
    rows_per_s = N_PAD // ns
    zcopies = rows_per_s // CHUNK
    dcol = d // 16

    assert n_chunks % 2 == 1

    out_type = [jax.ShapeDtypeStruct((nc, N_PAD, d), jnp.float32)]
    scratch = [
        pltpu.VMEM((CHUNK, d), jnp.float32),
        pltpu.VMEM((CHUNK, d), jnp.float32),
        pltpu.VMEM((CHUNK,), jnp.int32),
        pltpu.VMEM((CHUNK,), jnp.int32),
        pltpu.VMEM((n_chunks, CHUNK), jnp.int32),
        pltpu.VMEM_SHARED((N_PAD, d), jnp.float32),
        pltpu.SemaphoreType.DMA,
        pltpu.SemaphoreType.DMA,
        pltpu.SemaphoreType.DMA,
        pltpu.SemaphoreType.DMA,
    ]
    if with_counts:
        out_type.append(jax.ShapeDtypeStruct((nc, ns, N_PAD), jnp.float32))
        scratch.append(pltpu.VMEM((N_PAD,), jnp.float32))

    mesh = plsc.VectorSubcoreMesh(core_axis_name="c", subcore_axis_name="s")

    @functools.partial(
        pl.kernel, mesh=mesh, out_type=tuple(out_type),
        scratch_types=scratch,
        compiler_params=pltpu.CompilerParams(needs_layout_passes=False))
    def k(table_hbm, src_hbm, dst3_hbm, *refs):
        if with_counts:
            (agg_out, cnt_out, rb0, rb1, sb0, sb1, didx2, acc, sem0, sem1,
             isem0, isem1, cntbuf) = refs
        else:
            (agg_out, rb0, rb1, sb0, sb1, didx2, acc, sem0, sem1,
             isem0, isem1) = refs
            cnt_out = cntbuf = None
        bufs = (rb0, rb1)
        sbufs = (sb0, sb1)
        sems = (sem0, sem1)
        isems = (isem0, isem1)
        cid = lax.axis_index("c")
        sid = lax.axis_index("s")
        wid = sid * nc + cid
        row0 = sid * rows_per_s
        ebase = wid * e_per_w

        pltpu.sync_copy(dst3_hbm.at[wid], didx2)

        def zrow(i, c):
            rb0[i // dcol, pl.ds((i % dcol) * 16, 16)] = jnp.zeros(
                (16,), jnp.float32)
            return c
        lax.fori_loop(0, CHUNK * dcol, zrow, 0)
        for t in range(zcopies):
            pltpu.sync_copy(rb0, acc.at[pl.ds(row0 + t * CHUNK, CHUNK)])
        if with_counts:
            def zcnt(i, c):
                cntbuf[pl.ds(i * 16, 16)] = jnp.zeros((16,), jnp.float32)
                return c
            lax.fori_loop(0, N_PAD // 16, zcnt, 0)
        plsc.subcore_barrier()

        def sload(t, b):
            off = pl.multiple_of(ebase + t * CHUNK, 8)
            return pltpu.make_async_copy(
                src_hbm.at[pl.ds(off, CHUNK)], sbufs[b], isems[b])

        def gather(t, b):
            return pltpu.make_async_copy(
                table_hbm.at[sbufs[b]], bufs[b], sems[b])

        sload(0, 0).start()
        sload(0, 0).wait()
        gather(0, 0).start()
        sload(1, 1).start()
        sload(1, 1).wait()
        gather(1, 1).start()

        def pair_body(i, c):
            for b in range(2):
                t = 2 * i + b
                gather(t, b).wait()

                @pl.when(t + 2 < n_chunks)
                def _():
                    sload(t + 2, b).start()
                pltpu.sync_copy(bufs[b], acc.at[didx2.at[t]], add=True)
                if with_counts:
                    for j in range(CHUNK // 16):
                        dv = didx2[t, pl.ds(j * 16, 16)]
                        plsc.addupdate_scatter(
                            cntbuf, [dv], jnp.full((16,), 1.0, jnp.float32))

                @pl.when(t + 2 < n_chunks)
                def _():
                    sload(t + 2, b).wait()
                    gather(t + 2, b).start()
            return c
        lax.fori_loop(0, n_chunks // 2, pair_body, 0)
        t_last = n_chunks - 1
        gather(t_last, 0).wait()
        pltpu.sync_copy(bufs[0], acc.at[didx2.at[t_last]], add=True)
        if with_counts:
            for j in range(CHUNK // 16):
                dv = didx2[t_last, pl.ds(j * 16, 16)]
                plsc.addupdate_scatter(
                    cntbuf, [dv], jnp.full((16,), 1.0, jnp.float32))
        plsc.subcore_barrier()

        pltpu.sync_copy(acc.at[pl.ds(row0, rows_per_s)],
                        agg_out.at[cid, pl.ds(row0, rows_per_s)])
        if with_counts:
            pltpu.sync_copy(cntbuf, cnt_out.at[cid, sid])

    return k


_BR = 1000


def _tc_mid(aggp, cntp, x, w1l, w1r, b1, w2r, b2):
    nb = N_NODES // _BR

    def body(aggp_ref, cntp_ref, x_ref, w1l_ref, w1r_ref, b1_ref,
             w2r_ref, b2_ref, h_ref, z_ref):
        agg = aggp_ref[0] + aggp_ref[1]
        cnt = jnp.sum(cntp_ref[...], axis=1)[:, None]
        mean = agg / jnp.maximum(cnt, 1.0)
        h = jnp.maximum(
            jnp.dot(mean, w1l_ref[...], preferred_element_type=jnp.float32)
            + jnp.dot(x_ref[...], w1r_ref[...],
                      preferred_element_type=jnp.float32)
            + b1_ref[...], 0.0)
        h_ref[...] = h
        z_ref[...] = jnp.dot(h, w2r_ref[...],
                             preferred_element_type=jnp.float32) + b2_ref[...]

    nsub = cntp.shape[1]
    return pl.pallas_call(
        body,
        grid=(nb,),
        in_specs=[
            pl.BlockSpec((2, _BR, D_FEAT), lambda i: (0, i, 0)),
            pl.BlockSpec((_BR, nsub), lambda i: (i, 0)),
            pl.BlockSpec((_BR, D_FEAT), lambda i: (i, 0)),
            pl.BlockSpec((D_FEAT, HIDDEN), lambda i: (0, 0)),
            pl.BlockSpec((D_FEAT, HIDDEN), lambda i: (0, 0)),
            pl.BlockSpec((1, HIDDEN), lambda i: (0, 0)),
            pl.BlockSpec((HIDDEN, N_CLASSES), lambda i: (0, 0)),
            pl.BlockSpec((1, N_CLASSES), lambda i: (0, 0)),
        ],
        out_specs=[
            pl.BlockSpec((_BR, HIDDEN), lambda i: (i, 0)),
            pl.BlockSpec((_BR, N_CLASSES), lambda i: (i, 0)),
        ],
        out_shape=[
            jax.ShapeDtypeStruct((N_NODES, HIDDEN), jnp.float32),
            jax.ShapeDtypeStruct((N_NODES, N_CLASSES), jnp.float32),
        ],
    )(aggp, cntp, x, w1l, w1r, b1, w2r, b2)


def _tc_final(agg2p, cntp, z, w2l):
    nb = N_NODES // _BR

    def body(a2_ref, cntp_ref, z_ref, w2l_ref, o_ref):
        s = a2_ref[0] + a2_ref[1]
        cnt = jnp.sum(cntp_ref[...], axis=1)[:, None]
        mean = s / jnp.maximum(cnt, 1.0)
        o_ref[...] = jnp.dot(mean, w2l_ref[...],
                             preferred_element_type=jnp.float32) + z_ref[...]

    nsub = cntp.shape[1]
    return pl.pallas_call(
        body,
        grid=(nb,),
        in_specs=[
            pl.BlockSpec((2, _BR, HIDDEN), lambda i: (0, i, 0)),
            pl.BlockSpec((_BR, nsub), lambda i: (i, 0)),
            pl.BlockSpec((_BR, N_CLASSES), lambda i: (i, 0)),
            pl.BlockSpec((HIDDEN, N_CLASSES), lambda i: (0, 0)),
        ],
        out_specs=pl.BlockSpec((_BR, N_CLASSES), lambda i: (i, 0)),
        out_shape=jax.ShapeDtypeStruct((N_NODES, N_CLASSES), jnp.float32),
    )(agg2p, cntp, z, w2l)


def kernel(x, edge_index, W1_l, W1_r, b1, W2_l, W2_r, b2):
    nw = N_EDGES // CHUNK // 125
    src = edge_index[0].astype(jnp.int32)
    dst = edge_index[1].astype(jnp.int32).reshape(nw, 125, CHUNK)

    sc1 = _make_sc_segment_sum(D_FEAT, True)
    aggp, cntp = sc1(x, src, dst)
    cntp = cntp.reshape(-1, N_PAD).T

    h, z = _tc_mid(aggp, cntp, x, W1_l, W1_r, b1.reshape(1, -1),
                   W2_r, b2.reshape(1, -1))

    sc2 = _make_sc_segment_sum(HIDDEN, False)
    agg2p = sc2(h, src, dst)
    if isinstance(agg2p, (tuple, list)):
        agg2p = agg2p[0]

    return _tc_final(agg2p, cntp, z, W2_l)


# speedup vs baseline: 12.4935x; 1.0335x over previous
import functools

import jax
import jax.numpy as jnp
from jax import lax
from jax.experimental import pallas as pl
from jax.experimental.pallas import tpu as pltpu
from jax.experimental.pallas import tpu_sc as plsc

N_NODES = 10000
N_PAD = 10240
D_FEAT = 128
HIDDEN = 128
N_CLASSES = 40
N_EDGES = 320000
CHUNK = 80


@functools.lru_cache(maxsize=None)
def _make_sc_segment_sum(d: int, with_counts: bool):
    info = plsc.get_sparse_core_info()
    nc, ns = info.num_cores, info.num_subcores
    nw = nc * ns
    e_per_w = N_EDGES // nw
    assert N_EDGES % nw == 0 and e_per_w % CHUNK == 0
    n_chunks = e_per_w // CHUNK
    rows_per_s = N_PAD // ns
    zcopies = rows_per_s // CHUNK
    dcol = d // 16

    NB = 3
    assert n_chunks % NB == 2

    out_type = [jax.ShapeDtypeStruct((nc, N_PAD, d), jnp.float32)]
    scratch = (
        [pltpu.VMEM((CHUNK, d), jnp.float32) for _ in range(NB)]
        + [pltpu.VMEM((CHUNK,), jnp.int32) for _ in range(NB)]
        + [pltpu.VMEM((CHUNK,), jnp.int32) for _ in range(NB)]
        + [pltpu.VMEM_SHARED((N_PAD, d), jnp.float32)]
        + [pltpu.SemaphoreType.DMA] * (4 * NB)
    )
    if with_counts:
        out_type.append(jax.ShapeDtypeStruct((nc, ns, N_PAD), jnp.float32))
        scratch.append(pltpu.VMEM((N_PAD,), jnp.float32))

    mesh = plsc.VectorSubcoreMesh(core_axis_name="c", subcore_axis_name="s")

    @functools.partial(
        pl.kernel, mesh=mesh, out_type=tuple(out_type),
        scratch_types=scratch,
        compiler_params=pltpu.CompilerParams(needs_layout_passes=False))
    def k(table_hbm, src_hbm, dst_hbm, *refs):
        if with_counts:
            agg_out, cnt_out, *rest = refs
            cntbuf = rest[-1]
            rest = rest[:-1]
        else:
            agg_out, *rest = refs
            cnt_out = cntbuf = None
        rbufs = rest[0:NB]
        sbufs = rest[NB:2 * NB]
        dbufs = rest[2 * NB:3 * NB]
        acc = rest[3 * NB]
        gsems = rest[3 * NB + 1:3 * NB + 1 + NB]
        isems = rest[3 * NB + 1 + NB:3 * NB + 1 + 2 * NB]
        jsems = rest[3 * NB + 1 + 2 * NB:3 * NB + 1 + 3 * NB]
        ssems = rest[3 * NB + 1 + 3 * NB:3 * NB + 1 + 4 * NB]
        cid = lax.axis_index("c")
        sid = lax.axis_index("s")
        wid = sid * nc + cid
        row0 = sid * rows_per_s
        ebase = wid * e_per_w

        def zrow(i, c):
            rbufs[0][i // dcol, pl.ds((i % dcol) * 16, 16)] = jnp.zeros(
                (16,), jnp.float32)
            return c
        lax.fori_loop(0, CHUNK * dcol, zrow, 0)
        for t in range(zcopies):
            pltpu.sync_copy(rbufs[0], acc.at[pl.ds(row0 + t * CHUNK, CHUNK)])
        if with_counts:
            def zcnt(i, c):
                cntbuf[pl.ds(i * 16, 16)] = jnp.zeros((16,), jnp.float32)
                return c
            lax.fori_loop(0, N_PAD // 16, zcnt, 0)
        plsc.subcore_barrier()

        def sload(t, b):
            off = pl.multiple_of(ebase + t * CHUNK, 8)
            return pltpu.make_async_copy(
                src_hbm.at[pl.ds(off, CHUNK)], sbufs[b], isems[b])

        def dload(t, b):
            off = pl.multiple_of(ebase + t * CHUNK, 8)
            return pltpu.make_async_copy(
                dst_hbm.at[pl.ds(off, CHUNK)], dbufs[b], jsems[b])

        def gath(b):
            return pltpu.make_async_copy(
                table_hbm.at[sbufs[b]], rbufs[b], gsems[b])

        def scat(b):
            return pltpu.make_async_copy(rbufs[b], acc.at[dbufs[b]],
                                         ssems[b])

        def counts(b):
            if with_counts:
                for j in range(CHUNK // 16):
                    dv = dbufs[b][pl.ds(j * 16, 16)]
                    plsc.addupdate_scatter(
                        cntbuf, [dv], jnp.full((16,), 1.0, jnp.float32))

        for t in (0, 1):
            sload(t, t).start()
            dload(t, t).start()
            sload(t, t).wait()
            gath(t).start()
        sload(2, 2).start()

        def step(t, b, has_next3, has_next2, has_prev):
            gath(b).wait()
            if has_next3:
                sload(t + 3, b).start()
            dload(t, b).wait()
            scat(b).start(add=True)
            counts(b)
            scat(b).wait()
            if has_next2:
                b2 = (b + 2) % NB
                dload(t + 2, b2).start()
                sload(t + 2, b2).wait()
                gath(b2).start()

        def tri_body(i, c):
            for b in range(NB):
                t3 = i * NB + b
                step(t3, b, True, True, True)
            return c
        for b in range(NB):
            step(b, b, True, True, b > 0)
        lax.fori_loop(1, n_chunks // NB, tri_body, 0)
        tail0 = n_chunks - 2
        step(tail0, tail0 % NB, False, False, False)
        step(tail0 + 1, (tail0 + 1) % NB, False, False, False)
        plsc.subcore_barrier()

        pltpu.sync_copy(acc.at[pl.ds(row0, rows_per_s)],
                        agg_out.at[cid, pl.ds(row0, rows_per_s)])
        if with_counts:
            pltpu.sync_copy(cntbuf, cnt_out.at[cid, sid])

    return k


_BR = 1000


def _tc_mid(aggp, cntp, x, w1l, w1r, b1, w2r, b2):
    nb = N_NODES // _BR

    def body(aggp_ref, cntp_ref, x_ref, w1l_ref, w1r_ref, b1_ref,
             w2r_ref, b2_ref, h_ref, z_ref):
        agg = aggp_ref[0] + aggp_ref[1]
        cnt = jnp.sum(cntp_ref[...], axis=1)[:, None]
        mean = agg / jnp.maximum(cnt, 1.0)
        h = jnp.maximum(
            jnp.dot(mean, w1l_ref[...], preferred_element_type=jnp.float32)
            + jnp.dot(x_ref[...], w1r_ref[...],
                      preferred_element_type=jnp.float32)
            + b1_ref[...], 0.0)
        h_ref[...] = h
        z_ref[...] = jnp.dot(h, w2r_ref[...],
                             preferred_element_type=jnp.float32) + b2_ref[...]

    nsub = cntp.shape[1]
    return pl.pallas_call(
        body,
        grid=(nb,),
        in_specs=[
            pl.BlockSpec((2, _BR, D_FEAT), lambda i: (0, i, 0)),
            pl.BlockSpec((_BR, nsub), lambda i: (i, 0)),
            pl.BlockSpec((_BR, D_FEAT), lambda i: (i, 0)),
            pl.BlockSpec((D_FEAT, HIDDEN), lambda i: (0, 0)),
            pl.BlockSpec((D_FEAT, HIDDEN), lambda i: (0, 0)),
            pl.BlockSpec((1, HIDDEN), lambda i: (0, 0)),
            pl.BlockSpec((HIDDEN, N_CLASSES), lambda i: (0, 0)),
            pl.BlockSpec((1, N_CLASSES), lambda i: (0, 0)),
        ],
        out_specs=[
            pl.BlockSpec((_BR, HIDDEN), lambda i: (i, 0)),
            pl.BlockSpec((_BR, N_CLASSES), lambda i: (i, 0)),
        ],
        out_shape=[
            jax.ShapeDtypeStruct((N_NODES, HIDDEN), jnp.float32),
            jax.ShapeDtypeStruct((N_NODES, N_CLASSES), jnp.float32),
        ],
    )(aggp, cntp, x, w1l, w1r, b1, w2r, b2)


def _tc_final(agg2p, cntp, z, w2l):
    nb = N_NODES // _BR

    def body(a2_ref, cntp_ref, z_ref, w2l_ref, o_ref):
        s = a2_ref[0] + a2_ref[1]
        cnt = jnp.sum(cntp_ref[...], axis=1)[:, None]
        mean = s / jnp.maximum(cnt, 1.0)
        o_ref[...] = jnp.dot(mean, w2l_ref[...],
                             preferred_element_type=jnp.float32) + z_ref[...]

    nsub = cntp.shape[1]
    return pl.pallas_call(
        body,
        grid=(nb,),
        in_specs=[
            pl.BlockSpec((2, _BR, HIDDEN), lambda i: (0, i, 0)),
            pl.BlockSpec((_BR, nsub), lambda i: (i, 0)),
            pl.BlockSpec((_BR, N_CLASSES), lambda i: (i, 0)),
            pl.BlockSpec((HIDDEN, N_CLASSES), lambda i: (0, 0)),
        ],
        out_specs=pl.BlockSpec((_BR, N_CLASSES), lambda i: (i, 0)),
        out_shape=jax.ShapeDtypeStruct((N_NODES, N_CLASSES), jnp.float32),
    )(agg2p, cntp, z, w2l)


def kernel(x, edge_index, W1_l, W1_r, b1, W2_l, W2_r, b2):
    src = edge_index[0].astype(jnp.int32)
    dst = edge_index[1].astype(jnp.int32)

    sc1 = _make_sc_segment_sum(D_FEAT, True)
    aggp, cntp = sc1(x, src, dst)
    cntp = cntp.reshape(-1, N_PAD).T

    h, z = _tc_mid(aggp, cntp, x, W1_l, W1_r, b1.reshape(1, -1),
                   W2_r, b2.reshape(1, -1))

    sc2 = _make_sc_segment_sum(HIDDEN, False)
    agg2p = sc2(h, src, dst)
    if isinstance(agg2p, (tuple, list)):
        agg2p = agg2p[0]

    return _tc_final(agg2p, cntp, z, W2_l)


# speedup vs baseline: 13.1360x; 1.0514x over previous
import functools

import jax
import jax.numpy as jnp
from jax import lax
from jax.experimental import pallas as pl
from jax.experimental.pallas import tpu as pltpu
from jax.experimental.pallas import tpu_sc as plsc

N_NODES = 10000
N_PAD = 10240
D_FEAT = 128
HIDDEN = 128
N_CLASSES = 40
N_EDGES = 320000
CHUNK = 80


@functools.lru_cache(maxsize=None)
def _make_sc_segment_sum(d: int, with_counts: bool):
    info = plsc.get_sparse_core_info()
    nc, ns = info.num_cores, info.num_subcores
    nw = nc * ns
    e_per_w = N_EDGES // nw
    assert N_EDGES % nw == 0 and e_per_w % CHUNK == 0
    n_chunks = e_per_w // CHUNK
    rows_per_s = N_PAD // ns
    zcopies = rows_per_s // CHUNK
    dcol = d // 16

    NB = 3
    assert n_chunks % NB == 2

    out_type = [jax.ShapeDtypeStruct((nc, N_PAD, d), jnp.float32)]
    scratch = (
        [pltpu.VMEM((CHUNK, d), jnp.float32) for _ in range(NB)]
        + [pltpu.VMEM((CHUNK,), jnp.int32) for _ in range(NB)]
        + [pltpu.VMEM((CHUNK,), jnp.int32) for _ in range(NB)]
        + [pltpu.VMEM_SHARED((N_PAD, d), jnp.float32)]
        + [pltpu.SemaphoreType.DMA] * (4 * NB)
    )
    if with_counts:
        out_type.append(jax.ShapeDtypeStruct((nc, ns, N_PAD), jnp.float32))
        scratch.append(pltpu.VMEM((N_PAD,), jnp.float32))

    mesh = plsc.VectorSubcoreMesh(core_axis_name="c", subcore_axis_name="s")

    @functools.partial(
        pl.kernel, mesh=mesh, out_type=tuple(out_type),
        scratch_types=scratch,
        compiler_params=pltpu.CompilerParams(needs_layout_passes=False))
    def k(table_hbm, edge_hbm, *refs):
        if with_counts:
            agg_out, cnt_out, *rest = refs
            cntbuf = rest[-1]
            rest = rest[:-1]
        else:
            agg_out, *rest = refs
            cnt_out = cntbuf = None
        rbufs = rest[0:NB]
        sbufs = rest[NB:2 * NB]
        dbufs = rest[2 * NB:3 * NB]
        acc = rest[3 * NB]
        gsems = rest[3 * NB + 1:3 * NB + 1 + NB]
        isems = rest[3 * NB + 1 + NB:3 * NB + 1 + 2 * NB]
        jsems = rest[3 * NB + 1 + 2 * NB:3 * NB + 1 + 3 * NB]
        ssems = rest[3 * NB + 1 + 3 * NB:3 * NB + 1 + 4 * NB]
        cid = lax.axis_index("c")
        sid = lax.axis_index("s")
        wid = sid * nc + cid
        row0 = sid * rows_per_s
        ebase = wid * e_per_w

        def zrow(i, c):
            rbufs[0][i // dcol, pl.ds((i % dcol) * 16, 16)] = jnp.zeros(
                (16,), jnp.float32)
            return c
        lax.fori_loop(0, CHUNK * dcol, zrow, 0)
        for t in range(zcopies):
            pltpu.sync_copy(rbufs[0], acc.at[pl.ds(row0 + t * CHUNK, CHUNK)])
        if with_counts:
            def zcnt(i, c):
                cntbuf[pl.ds(i * 16, 16)] = jnp.zeros((16,), jnp.float32)
                return c
            lax.fori_loop(0, N_PAD // 16, zcnt, 0)
        plsc.subcore_barrier()

        def sload(t, b):
            off = pl.multiple_of(ebase + t * CHUNK, 8)
            return pltpu.make_async_copy(
                edge_hbm.at[pl.ds(off, CHUNK)], sbufs[b], isems[b])

        def dload(t, b):
            off = pl.multiple_of(N_EDGES + ebase + t * CHUNK, 8)
            return pltpu.make_async_copy(
                edge_hbm.at[pl.ds(off, CHUNK)], dbufs[b], jsems[b])

        def gath(b):
            return pltpu.make_async_copy(
                table_hbm.at[sbufs[b]], rbufs[b], gsems[b])

        def scat(b):
            return pltpu.make_async_copy(rbufs[b], acc.at[dbufs[b]],
                                         ssems[b])

        def counts(b):
            if with_counts:
                for j in range(CHUNK // 16):
                    dv = dbufs[b][pl.ds(j * 16, 16)]
                    plsc.addupdate_scatter(
                        cntbuf, [dv], jnp.full((16,), 1.0, jnp.float32))

        for t in (0, 1):
            sload(t, t).start()
            dload(t, t).start()
            sload(t, t).wait()
            gath(t).start()
        sload(2, 2).start()

        def step(t, b, has_next3, has_next2, has_prev):
            gath(b).wait()
            if has_next3:
                sload(t + 3, b).start()
            dload(t, b).wait()
            scat(b).start(add=True)
            counts(b)
            scat(b).wait()
            if has_next2:
                b2 = (b + 2) % NB
                dload(t + 2, b2).start()
                sload(t + 2, b2).wait()
                gath(b2).start()

        def tri_body(i, c):
            for b in range(NB):
                t3 = i * NB + b
                step(t3, b, True, True, True)
            return c
        for b in range(NB):
            step(b, b, True, True, b > 0)
        lax.fori_loop(1, n_chunks // NB, tri_body, 0)
        tail0 = n_chunks - 2
        step(tail0, tail0 % NB, False, False, False)
        step(tail0 + 1, (tail0 + 1) % NB, False, False, False)
        plsc.subcore_barrier()

        pltpu.sync_copy(acc.at[pl.ds(row0, rows_per_s)],
                        agg_out.at[cid, pl.ds(row0, rows_per_s)])
        if with_counts:
            pltpu.sync_copy(cntbuf, cnt_out.at[cid, sid])

    return k


_BR = 2000


def _tc_mid(aggp, cntp, x, w1l, w1r, b1, w2r, b2):
    nb = N_NODES // _BR

    def body(aggp_ref, cntp_ref, x_ref, w1l_ref, w1r_ref, b1_ref,
             w2r_ref, b2_ref, h_ref, z_ref):
        agg = aggp_ref[0] + aggp_ref[1]
        cnt = jnp.sum(cntp_ref[...], axis=1)[:, None]
        mean = agg / jnp.maximum(cnt, 1.0)
        h = jnp.maximum(
            jnp.dot(mean, w1l_ref[...], preferred_element_type=jnp.float32)
            + jnp.dot(x_ref[...], w1r_ref[...],
                      preferred_element_type=jnp.float32)
            + b1_ref[...], 0.0)
        h_ref[...] = h
        z_ref[...] = jnp.dot(h, w2r_ref[...],
                             preferred_element_type=jnp.float32) + b2_ref[...]

    nsub = cntp.shape[1]
    return pl.pallas_call(
        body,
        grid=(nb,),
        in_specs=[
            pl.BlockSpec((2, _BR, D_FEAT), lambda i: (0, i, 0)),
            pl.BlockSpec((_BR, nsub), lambda i: (i, 0)),
            pl.BlockSpec((_BR, D_FEAT), lambda i: (i, 0)),
            pl.BlockSpec((D_FEAT, HIDDEN), lambda i: (0, 0)),
            pl.BlockSpec((D_FEAT, HIDDEN), lambda i: (0, 0)),
            pl.BlockSpec((1, HIDDEN), lambda i: (0, 0)),
            pl.BlockSpec((HIDDEN, N_CLASSES), lambda i: (0, 0)),
            pl.BlockSpec((1, N_CLASSES), lambda i: (0, 0)),
        ],
        out_specs=[
            pl.BlockSpec((_BR, HIDDEN), lambda i: (i, 0)),
            pl.BlockSpec((_BR, N_CLASSES), lambda i: (i, 0)),
        ],
        out_shape=[
            jax.ShapeDtypeStruct((N_NODES, HIDDEN), jnp.float32),
            jax.ShapeDtypeStruct((N_NODES, N_CLASSES), jnp.float32),
        ],
    )(aggp, cntp, x, w1l, w1r, b1, w2r, b2)


def _tc_final(agg2p, cntp, z, w2l):
    nb = N_NODES // _BR

    def body(a2_ref, cntp_ref, z_ref, w2l_ref, o_ref):
        s = a2_ref[0] + a2_ref[1]
        cnt = jnp.sum(cntp_ref[...], axis=1)[:, None]
        mean = s / jnp.maximum(cnt, 1.0)
        o_ref[...] = jnp.dot(mean, w2l_ref[...],
                             preferred_element_type=jnp.float32) + z_ref[...]

    nsub = cntp.shape[1]
    return pl.pallas_call(
        body,
        grid=(nb,),
        in_specs=[
            pl.BlockSpec((2, _BR, HIDDEN), lambda i: (0, i, 0)),
            pl.BlockSpec((_BR, nsub), lambda i: (i, 0)),
            pl.BlockSpec((_BR, N_CLASSES), lambda i: (i, 0)),
            pl.BlockSpec((HIDDEN, N_CLASSES), lambda i: (0, 0)),
        ],
        out_specs=pl.BlockSpec((_BR, N_CLASSES), lambda i: (i, 0)),
        out_shape=jax.ShapeDtypeStruct((N_NODES, N_CLASSES), jnp.float32),
    )(agg2p, cntp, z, w2l)


def kernel(x, edge_index, W1_l, W1_r, b1, W2_l, W2_r, b2):
    ei = edge_index.astype(jnp.int32).reshape(2 * N_EDGES)

    sc1 = _make_sc_segment_sum(D_FEAT, True)
    aggp, cntp = sc1(x, ei)
    cntp = cntp.reshape(-1, N_PAD).T

    h, z = _tc_mid(aggp, cntp, x, W1_l, W1_r, b1.reshape(1, -1),
                   W2_r, b2.reshape(1, -1))

    sc2 = _make_sc_segment_sum(HIDDEN, False)
    agg2p = sc2(h, ei)
    if isinstance(agg2p, (tuple, list)):
        agg2p = agg2p[0]

    return _tc_final(agg2p, cntp, z, W2_l)


# speedup vs baseline: 15.0282x; 1.1440x over previous
import functools

import jax
import jax.numpy as jnp
from jax import lax
from jax.experimental import pallas as pl
from jax.experimental.pallas import tpu as pltpu
from jax.experimental.pallas import tpu_sc as plsc

N_NODES = 10000
N_PAD = 10240
D_FEAT = 128
HIDDEN = 128
N_CLASSES = 40
N_EDGES = 320000
CHUNK = 80


@functools.lru_cache(maxsize=None)
def _make_sc_segment_sum(d: int, with_counts: bool):
    info = plsc.get_sparse_core_info()
    nc, ns = info.num_cores, info.num_subcores
    nw = nc * ns
    e_per_w = N_EDGES // nw
    assert N_EDGES % nw == 0 and e_per_w % CHUNK == 0
    n_chunks = e_per_w // CHUNK
    rows_per_s = N_PAD // ns
    zcopies = rows_per_s // CHUNK
    dcol = d // 16

    NB = 3
    assert n_chunks % NB == 2

    out_type = [jax.ShapeDtypeStruct((nc, N_PAD, d), jnp.float32)]
    scratch = (
        [pltpu.VMEM((CHUNK, d), jnp.float32) for _ in range(NB)]
        + [pltpu.VMEM((CHUNK,), jnp.int32) for _ in range(NB)]
        + [pltpu.VMEM((CHUNK,), jnp.int32) for _ in range(NB)]
        + [pltpu.VMEM_SHARED((N_PAD, d), jnp.float32)]
        + [pltpu.SemaphoreType.DMA] * (4 * NB)
    )
    if with_counts:
        out_type.append(jax.ShapeDtypeStruct((nc, ns, N_PAD), jnp.float32))
        scratch.append(pltpu.VMEM((N_PAD,), jnp.float32))

    mesh = plsc.VectorSubcoreMesh(core_axis_name="c", subcore_axis_name="s")

    @functools.partial(
        pl.kernel, mesh=mesh, out_type=tuple(out_type),
        scratch_types=scratch,
        compiler_params=pltpu.CompilerParams(needs_layout_passes=False))
    def k(table_hbm, edge_hbm, *refs):
        if with_counts:
            agg_out, cnt_out, *rest = refs
            cntbuf = rest[-1]
            rest = rest[:-1]
        else:
            agg_out, *rest = refs
            cnt_out = cntbuf = None
        rbufs = rest[0:NB]
        sbufs = rest[NB:2 * NB]
        dbufs = rest[2 * NB:3 * NB]
        acc = rest[3 * NB]
        gsems = rest[3 * NB + 1:3 * NB + 1 + NB]
        isems = rest[3 * NB + 1 + NB:3 * NB + 1 + 2 * NB]
        jsems = rest[3 * NB + 1 + 2 * NB:3 * NB + 1 + 3 * NB]
        ssems = rest[3 * NB + 1 + 3 * NB:3 * NB + 1 + 4 * NB]
        cid = lax.axis_index("c")
        sid = lax.axis_index("s")
        wid = sid * nc + cid
        row0 = sid * rows_per_s
        ebase = wid * e_per_w

        def zrow(i, c):
            rbufs[0][i // dcol, pl.ds((i % dcol) * 16, 16)] = jnp.zeros(
                (16,), jnp.float32)
            return c
        lax.fori_loop(0, CHUNK * dcol, zrow, 0)
        for t in range(zcopies):
            pltpu.sync_copy(rbufs[0], acc.at[pl.ds(row0 + t * CHUNK, CHUNK)])
        if with_counts:
            def zcnt(i, c):
                cntbuf[pl.ds(i * 16, 16)] = jnp.zeros((16,), jnp.float32)
                return c
            lax.fori_loop(0, N_PAD // 16, zcnt, 0)
        plsc.subcore_barrier()

        def sload(t, b):
            off = pl.multiple_of(ebase + t * CHUNK, 8)
            return pltpu.make_async_copy(
                edge_hbm.at[pl.ds(off, CHUNK)], sbufs[b], isems[b])

        def dload(t, b):
            off = pl.multiple_of(N_EDGES + ebase + t * CHUNK, 8)
            return pltpu.make_async_copy(
                edge_hbm.at[pl.ds(off, CHUNK)], dbufs[b], jsems[b])

        def gath(b):
            return pltpu.make_async_copy(
                table_hbm.at[sbufs[b]], rbufs[b], gsems[b])

        def scat(b):
            return pltpu.make_async_copy(rbufs[b], acc.at[dbufs[b]],
                                         ssems[b])

        def counts(b):
            if with_counts:
                for j in range(CHUNK // 16):
                    dv = dbufs[b][pl.ds(j * 16, 16)]
                    plsc.addupdate_scatter(
                        cntbuf, [dv], jnp.full((16,), 1.0, jnp.float32))

        for t in (0, 1):
            sload(t, t).start()
            dload(t, t).start()
            sload(t, t).wait()
            gath(t).start()
        sload(2, 2).start()

        def step(t, b, has_next3, has_next2, has_prev):
            gath(b).wait()
            dload(t, b).wait()
            scat(b).start(add=True)
            if has_next3:
                sload(t + 3, b).start()
            if has_next2:
                b2 = (b + 2) % NB
                dload(t + 2, b2).start()
                sload(t + 2, b2).wait()
                gath(b2).start()
            counts(b)
            scat(b).wait()

        def tri_body(i, c):
            for b in range(NB):
                t3 = i * NB + b
                step(t3, b, True, True, True)
            return c
        for b in range(NB):
            step(b, b, True, True, b > 0)
        lax.fori_loop(1, n_chunks // NB, tri_body, 0)
        tail0 = n_chunks - 2
        step(tail0, tail0 % NB, False, False, False)
        step(tail0 + 1, (tail0 + 1) % NB, False, False, False)
        plsc.subcore_barrier()

        pltpu.sync_copy(acc.at[pl.ds(row0, rows_per_s)],
                        agg_out.at[cid, pl.ds(row0, rows_per_s)])
        if with_counts:
            pltpu.sync_copy(cntbuf, cnt_out.at[cid, sid])

    return k


_BR = 2000


def _tc_mid(aggp, cntp, x, w1l, w1r, b1, w2r, b2):
    nb = N_NODES // _BR

    def body(aggp_ref, cntp_ref, x_ref, w1l_ref, w1r_ref, b1_ref,
             w2r_ref, b2_ref, h_ref, z_ref):
        agg = aggp_ref[0] + aggp_ref[1]
        cnt = jnp.sum(cntp_ref[...], axis=1)[:, None]
        mean = agg / jnp.maximum(cnt, 1.0)
        h = jnp.maximum(
            jnp.dot(mean, w1l_ref[...], preferred_element_type=jnp.float32)
            + jnp.dot(x_ref[...], w1r_ref[...],
                      preferred_element_type=jnp.float32)
            + b1_ref[...], 0.0)
        h_ref[...] = h
        z_ref[...] = jnp.dot(h, w2r_ref[...],
                             preferred_element_type=jnp.float32) + b2_ref[...]

    nsub = cntp.shape[1]
    return pl.pallas_call(
        body,
        grid=(nb,),
        in_specs=[
            pl.BlockSpec((2, _BR, D_FEAT), lambda i: (0, i, 0)),
            pl.BlockSpec((_BR, nsub), lambda i: (i, 0)),
            pl.BlockSpec((_BR, D_FEAT), lambda i: (i, 0)),
            pl.BlockSpec((D_FEAT, HIDDEN), lambda i: (0, 0)),
            pl.BlockSpec((D_FEAT, HIDDEN), lambda i: (0, 0)),
            pl.BlockSpec((1, HIDDEN), lambda i: (0, 0)),
            pl.BlockSpec((HIDDEN, N_CLASSES), lambda i: (0, 0)),
            pl.BlockSpec((1, N_CLASSES), lambda i: (0, 0)),
        ],
        out_specs=[
            pl.BlockSpec((_BR, HIDDEN), lambda i: (i, 0)),
            pl.BlockSpec((_BR, N_CLASSES), lambda i: (i, 0)),
        ],
        out_shape=[
            jax.ShapeDtypeStruct((N_NODES, HIDDEN), jnp.float32),
            jax.ShapeDtypeStruct((N_NODES, N_CLASSES), jnp.float32),
        ],
    )(aggp, cntp, x, w1l, w1r, b1, w2r, b2)


def _tc_final(agg2p, cntp, z, w2l):
    nb = N_NODES // _BR

    def body(a2_ref, cntp_ref, z_ref, w2l_ref, o_ref):
        s = a2_ref[0] + a2_ref[1]
        cnt = jnp.sum(cntp_ref[...], axis=1)[:, None]
        mean = s / jnp.maximum(cnt, 1.0)
        o_ref[...] = jnp.dot(mean, w2l_ref[...],
                             preferred_element_type=jnp.float32) + z_ref[...]

    nsub = cntp.shape[1]
    return pl.pallas_call(
        body,
        grid=(nb,),
        in_specs=[
            pl.BlockSpec((2, _BR, HIDDEN), lambda i: (0, i, 0)),
            pl.BlockSpec((_BR, nsub), lambda i: (i, 0)),
            pl.BlockSpec((_BR, N_CLASSES), lambda i: (i, 0)),
            pl.BlockSpec((HIDDEN, N_CLASSES), lambda i: (0, 0)),
        ],
        out_specs=pl.BlockSpec((_BR, N_CLASSES), lambda i: (i, 0)),
        out_shape=jax.ShapeDtypeStruct((N_NODES, N_CLASSES), jnp.float32),
    )(agg2p, cntp, z, w2l)


def kernel(x, edge_index, W1_l, W1_r, b1, W2_l, W2_r, b2):
    ei = edge_index.astype(jnp.int32).reshape(2 * N_EDGES)

    sc1 = _make_sc_segment_sum(D_FEAT, True)
    aggp, cntp = sc1(x, ei)
    cntp = cntp.reshape(-1, N_PAD).T

    h, z = _tc_mid(aggp, cntp, x, W1_l, W1_r, b1.reshape(1, -1),
                   W2_r, b2.reshape(1, -1))

    sc2 = _make_sc_segment_sum(HIDDEN, False)
    agg2p = sc2(h, ei)
    if isinstance(agg2p, (tuple, list)):
        agg2p = agg2p[0]

    return _tc_final(agg2p, cntp, z, W2_l)
